# baseline (speedup 1.0000x reference)
import functools
import math

import jax
import jax.numpy as jnp
from jax.experimental import pallas as pl
from jax.experimental.pallas import tpu as pltpu

VOCAB = 100000
SMOOTHING = 0.1
PADDING_IDX = 0
CONFIDENCE = 1.0 - SMOOTHING
SV = SMOOTHING / (VOCAB - 2)
C_ENT = SMOOTHING * math.log(SV) + CONFIDENCE * math.log(CONFIDENCE)

BR = 8


def _loss_kernel(tgt_ref, pred_ref, out_ref, loss_ref, cnt_ref, *, nsteps,
                 vocab):
    i = pl.program_id(0)
    x = pred_ref[...]
    tgt = tgt_ref[...]

    bmax = jnp.max(x, axis=1, keepdims=True)
    sexp = jnp.sum(jnp.exp(x - bmax), axis=1, keepdims=True)
    lse = bmax + jnp.log(sexp)
    sump = jnp.sum(x, axis=1, keepdims=True)

    cols = jax.lax.broadcasted_iota(jnp.int32, (1, vocab), 1)
    pt = jnp.sum(jnp.where(cols == tgt, x, 0.0), axis=1, keepdims=True)
    p0 = x[:, 0:1]

    nonpad = tgt != PADDING_IDX
    loss_rows = jnp.where(
        nonpad,
        C_ENT - SV * sump + SV * p0 + (SV - CONFIDENCE) * pt + lse,
        0.0,
    )

    @pl.when(i == 0)
    def _init():
        loss_ref[...] = jnp.zeros((1, 1), jnp.float32)
        cnt_ref[...] = jnp.zeros((1, 1), jnp.float32)

    loss_ref[...] += jnp.sum(loss_rows).reshape(1, 1)
    cnt_ref[...] += jnp.sum(nonpad.astype(jnp.float32)).reshape(1, 1)

    @pl.when(i == nsteps - 1)
    def _finish():
        out_ref[...] = loss_ref[...] / cnt_ref[...]


@jax.jit
def kernel(pred, target):
    n, vocab = pred.shape
    nsteps = n // BR
    tgt2 = target.reshape(n, 1)
    out = pl.pallas_call(
        functools.partial(_loss_kernel, nsteps=nsteps, vocab=vocab),
        grid=(nsteps,),
        in_specs=[
            pl.BlockSpec((BR, 1), lambda i: (i, 0)),
            pl.BlockSpec((BR, vocab), lambda i: (i, 0)),
        ],
        out_specs=pl.BlockSpec((1, 1), lambda i: (0, 0)),
        out_shape=jax.ShapeDtypeStruct((1, 1), jnp.float32),
        scratch_shapes=[pltpu.VMEM((1, 1), jnp.float32) for _ in range(2)],
    )(tgt2, pred)
    return out[0, 0]


# speedup vs baseline: 1.2546x; 1.2546x over previous
import functools
import math

import jax
import jax.numpy as jnp
from jax.experimental import pallas as pl
from jax.experimental.pallas import tpu as pltpu

VOCAB = 100000
SMOOTHING = 0.1
PADDING_IDX = 0
CONFIDENCE = 1.0 - SMOOTHING
SV = SMOOTHING / (VOCAB - 2)
C_ENT = SMOOTHING * math.log(SV) + CONFIDENCE * math.log(CONFIDENCE)

BR = 8
G = 4


def _loss_kernel(tgt_ref, *refs, nsteps, vocab):
    pred_refs = refs[:G]
    out_ref, loss_ref, cnt_ref = refs[G], refs[G + 1], refs[G + 2]
    i = pl.program_id(0)
    tgt_all = tgt_ref[...]

    @pl.when(i == 0)
    def _init():
        loss_ref[...] = jnp.zeros((1, 1), jnp.float32)
        cnt_ref[...] = jnp.zeros((1, 1), jnp.float32)

    cols = jax.lax.broadcasted_iota(jnp.int32, (1, vocab), 1)
    for g in range(G):
        x = pred_refs[g][...]
        tgt = tgt_all[g * BR:(g + 1) * BR, :]

        bmax = jnp.max(x, axis=1, keepdims=True)
        sexp = jnp.sum(jnp.exp(x - bmax), axis=1, keepdims=True)
        lse = bmax + jnp.log(sexp)
        sump = jnp.sum(x, axis=1, keepdims=True)
        pt = jnp.sum(jnp.where(cols == tgt, x, 0.0), axis=1, keepdims=True)
        p0 = x[:, 0:1]

        nonpad = tgt != PADDING_IDX
        loss_rows = jnp.where(
            nonpad,
            C_ENT - SV * sump + SV * p0 + (SV - CONFIDENCE) * pt + lse,
            0.0,
        )
        loss_ref[...] += jnp.sum(loss_rows).reshape(1, 1)
        cnt_ref[...] += jnp.sum(nonpad.astype(jnp.float32)).reshape(1, 1)

    @pl.when(i == nsteps - 1)
    def _finish():
        out_ref[...] = loss_ref[...] / cnt_ref[...]


@jax.jit
def kernel(pred, target):
    n, vocab = pred.shape
    nsteps = n // (BR * G)
    tgt2 = target.reshape(n, 1)
    pred_spec = [
        pl.BlockSpec((BR, vocab), functools.partial(lambda g, i: (G * i + g, 0), g))
        for g in range(G)
    ]
    out = pl.pallas_call(
        functools.partial(_loss_kernel, nsteps=nsteps, vocab=vocab),
        grid=(nsteps,),
        in_specs=[pl.BlockSpec((G * BR, 1), lambda i: (i, 0))] + pred_spec,
        out_specs=pl.BlockSpec((1, 1), lambda i: (0, 0)),
        out_shape=jax.ShapeDtypeStruct((1, 1), jnp.float32),
        scratch_shapes=[pltpu.VMEM((1, 1), jnp.float32) for _ in range(2)],
    )(tgt2, *([pred] * G))
    return out[0, 0]
